# baseline (speedup 1.0000x reference)
import functools

import jax
import jax.numpy as jnp
from jax import lax
from jax.experimental import pallas as pl
from jax.experimental.pallas import tpu as pltpu
from jax.experimental.pallas import tpu_sc as plsc

H = 768
D_FF = 3072
S = 2048
NR, NS = 4, 2
FBLK = 1536
NF = D_FF // FBLK
KBLK = FBLK // 2
NEG = -1e30

NWORKERS = 32
TOKW = S // NWORKERS
LANES = 16


def _gelu(x):
    return 0.5 * x * (1.0 + jax.lax.erf(x * 0.7071067811865476))


def _router_core(x_ref, w1_ref, b1_ref, w2t_ref, b2t_ref, logits_ref, z_ref):
    x = x_ref[...]
    hr = _gelu(jnp.dot(x, w1_ref[...], preferred_element_type=jnp.float32)
               + b1_ref[...])
    logits = lax.dot_general(w2t_ref[...], hr, (((1,), (1,)), ((), ())),
                             preferred_element_type=jnp.float32) + b2t_ref[...]
    row = jax.lax.broadcasted_iota(jnp.int32, logits.shape, 0)
    logits = jnp.where(row < NR, logits, NEG)
    logits_ref[...] = logits
    m1 = jnp.max(logits, axis=0)
    lse = m1 + jnp.log(jnp.sum(jnp.exp(logits - m1[None, :]), axis=0))
    z_ref[...] = jnp.mean(jnp.square(lse)).reshape(1, 1)


def _sc_topk_body(logits_hbm, wmat_hbm, lg_v, wm_v):
    wid = lax.axis_index("s") * 2 + lax.axis_index("c")
    base = wid * TOKW
    for j in range(NR):
        pltpu.sync_copy(logits_hbm.at[pl.ds(j * S + base, TOKW)], lg_v.at[j])
    for i in range(TOKW // LANES):
        lsl = pl.ds(LANES * i, LANES)
        l = [lg_v[j, lsl] for j in range(NR)]
        m1 = jnp.maximum(jnp.maximum(l[0], l[1]), jnp.maximum(l[2], l[3]))
        idx1 = jnp.where(l[0] == m1, 0,
                         jnp.where(l[1] == m1, 1,
                                   jnp.where(l[2] == m1, 2, 3)))
        l2 = [jnp.where(idx1 == j, NEG, l[j]) for j in range(NR)]
        m2 = jnp.maximum(jnp.maximum(l2[0], l2[1]),
                         jnp.maximum(l2[2], l2[3]))
        idx2 = jnp.where(l2[0] == m2, 0,
                         jnp.where(l2[1] == m2, 1,
                                   jnp.where(l2[2] == m2, 2, 3)))
        e2 = jnp.exp(m2 - m1)
        wa = 1.0 / (1.0 + e2)
        wb = e2 * wa
        sl = pl.ds(LANES * i, LANES)
        for j in range(NR):
            wm_v[j, sl] = (jnp.where(idx1 == j, wa, 0.0)
                           + jnp.where(idx2 == j, wb, 0.0))
        half = jnp.full((LANES,), 1.0 / NS, jnp.float32)
        zero = jnp.zeros((LANES,), jnp.float32)
        for j in range(NR, NR + NS):
            wm_v[j, sl] = half
        for j in range(NR + NS, 8):
            wm_v[j, sl] = zero
    for j in range(8):
        pltpu.sync_copy(wm_v.at[j], wmat_hbm.at[pl.ds(j * S + base, TOKW)])


def _ffn_chunk(x, w1a_ref, w1b_ref, b1_ref, w2a_ref, w2b_ref):
    ha = _gelu(jnp.dot(x, w1a_ref[0], preferred_element_type=jnp.float32)
               + b1_ref[0, 0, :KBLK])
    hb = _gelu(jnp.dot(x, w1b_ref[0], preferred_element_type=jnp.float32)
               + b1_ref[0, 0, KBLK:])
    return (jnp.dot(ha, w2a_ref[0], preferred_element_type=jnp.float32)
            + jnp.dot(hb, w2b_ref[0], preferred_element_type=jnp.float32))


def _shared_body(x_ref, rw1_ref, rb1_ref, rw2t_ref, rb2t_ref,
                 w1a_ref, w1b_ref, b1_ref, w2a_ref, w2b_ref, b2_ref,
                 acc_ref, logits_ref, z_ref):
    g = pl.program_id(0)

    @pl.when(g == 0)
    def _router():
        _router_core(x_ref, rw1_ref, rb1_ref, rw2t_ref, rb2t_ref,
                     logits_ref, z_ref)

    @pl.when(g >= 1)
    def _expert():
        f = (g - 1) % NF
        delta = (1.0 / NS) * _ffn_chunk(x_ref[...], w1a_ref, w1b_ref,
                                        b1_ref, w2a_ref, w2b_ref)

        @pl.when(f == 0)
        def _bias():
            acc_ref[...] = jnp.where(g == 1, 0.0, acc_ref[...]) \
                + (1.0 / NS) * b2_ref[0, 0][None, :]

        acc_ref[...] += delta


def _routed_body(accs_ref, x_ref, wmat_ref,
                 w1a_ref, w1b_ref, b1_ref, w2a_ref, w2b_ref, b2_ref,
                 ow_ref, ob_ref, out_ref):
    g = pl.program_id(0)

    @pl.when(g == 0)
    def _init():
        out_ref[...] = accs_ref[...]

    @pl.when(g < NR * NF)
    def _expert():
        e = g // NF
        f = g % NF
        wcol = jnp.zeros((S,), jnp.float32)
        for j in range(NR):
            wcol = wcol + jnp.where(e == j, wmat_ref[j, :], 0.0)
        delta = wcol[:, None] * _ffn_chunk(x_ref[...], w1a_ref, w1b_ref,
                                           b1_ref, w2a_ref, w2b_ref)

        @pl.when(f == 0)
        def _bias():
            out_ref[...] += wcol[:, None] * b2_ref[0, 0][None, :]

        out_ref[...] += delta

    @pl.when(g == NR * NF)
    def _finish():
        o = jnp.dot(out_ref[...], ow_ref[...],
                    preferred_element_type=jnp.float32)
        o = (o + ob_ref[...]) * 0.5 + x_ref[...] * 0.5
        mean = jnp.mean(o, axis=-1, keepdims=True)
        o = o - mean
        var = jnp.mean(jnp.square(o), axis=-1, keepdims=True)
        out_ref[...] = o * jax.lax.rsqrt(var + 1e-6)


def _const_spec(shape):
    return pl.BlockSpec(shape, lambda *_: tuple(0 for _ in shape))


_sc_topk = functools.partial(
    pl.kernel,
    mesh=plsc.VectorSubcoreMesh(core_axis_name="c", subcore_axis_name="s"),
    out_type=jax.ShapeDtypeStruct((8 * S,), jnp.float32),
    scratch_types=[pltpu.VMEM((NR, TOKW), jnp.float32),
                   pltpu.VMEM((8, TOKW), jnp.float32)],
)(_sc_topk_body)


@functools.partial(jax.jit, static_argnames=("interpret",))
def _run(x2d, router_w1, router_b1, router_w2p, router_b2p,
         re_w1, re_b1, re_w2, re_b2,
         se_w1, se_b1, se_w2, se_b2, out_w, out_b, interpret=False):
    def sidx(g):
        gg = jnp.maximum(g - 1, 0)
        return gg // NF, gg % NF

    accs, logits, z_loss = pl.pallas_call(
        _shared_body,
        grid=(NS * NF + 1,),
        in_specs=[
            _const_spec((S, H)), _const_spec((H, H)),
            _const_spec((1, H)), _const_spec((8, H)), _const_spec((8, 1)),
            pl.BlockSpec((1, H, KBLK),
                         lambda g: (sidx(g)[0], 0, 2 * sidx(g)[1])),
            pl.BlockSpec((1, H, KBLK),
                         lambda g: (sidx(g)[0], 0, 2 * sidx(g)[1] + 1)),
            pl.BlockSpec((1, 1, FBLK), lambda g: (sidx(g)[0], 0, sidx(g)[1])),
            pl.BlockSpec((1, KBLK, H),
                         lambda g: (sidx(g)[0], 2 * sidx(g)[1], 0)),
            pl.BlockSpec((1, KBLK, H),
                         lambda g: (sidx(g)[0], 2 * sidx(g)[1] + 1, 0)),
            pl.BlockSpec((1, 1, H), lambda g: (sidx(g)[0], 0, 0)),
        ],
        out_specs=[_const_spec((S, H)), _const_spec((8, S)),
                   _const_spec((1, 1))],
        out_shape=[jax.ShapeDtypeStruct((S, H), jnp.float32),
                   jax.ShapeDtypeStruct((8, S), jnp.float32),
                   jax.ShapeDtypeStruct((1, 1), jnp.float32)],
        interpret=interpret,
    )(x2d, router_w1, router_b1.reshape(1, H), router_w2p, router_b2p,
      se_w1, se_w1, se_b1.reshape(NS, 1, D_FF),
      se_w2, se_w2, se_b2.reshape(NS, 1, H))

    wmat = _sc_topk(logits.reshape(-1)).reshape(8, S)

    def ridx(g):
        gg = jnp.minimum(g, NR * NF - 1)
        return gg // NF, gg % NF

    out = pl.pallas_call(
        _routed_body,
        grid=(NR * NF + 1,),
        in_specs=[
            _const_spec((S, H)), _const_spec((S, H)), _const_spec((8, S)),
            pl.BlockSpec((1, H, KBLK),
                         lambda g: (ridx(g)[0], 0, 2 * ridx(g)[1])),
            pl.BlockSpec((1, H, KBLK),
                         lambda g: (ridx(g)[0], 0, 2 * ridx(g)[1] + 1)),
            pl.BlockSpec((1, 1, FBLK), lambda g: (ridx(g)[0], 0, ridx(g)[1])),
            pl.BlockSpec((1, KBLK, H),
                         lambda g: (ridx(g)[0], 2 * ridx(g)[1], 0)),
            pl.BlockSpec((1, KBLK, H),
                         lambda g: (ridx(g)[0], 2 * ridx(g)[1] + 1, 0)),
            pl.BlockSpec((1, 1, H), lambda g: (ridx(g)[0], 0, 0)),
            _const_spec((H, H)), _const_spec((1, H)),
        ],
        out_specs=_const_spec((S, H)),
        out_shape=jax.ShapeDtypeStruct((S, H), jnp.float32),
        interpret=interpret,
    )(accs, x2d, wmat, re_w1, re_w1, re_b1.reshape(NR, 1, D_FF),
      re_w2, re_w2, re_b2.reshape(NR, 1, H), out_w, out_b.reshape(1, H))
    return out, z_loss


def kernel(hidden_states, router_w1, router_b1, router_w2, router_b2,
           re_w1, re_b1, re_w2, re_b2, se_w1, se_b1, se_w2, se_b2,
           out_w, out_b, interpret=False):
    x2d = hidden_states.reshape(S, H)
    router_w2p = jnp.pad(router_w2, ((0, 0), (0, 8 - NR))).T
    router_b2p = jnp.pad(router_b2, (0, 8 - NR)).reshape(8, 1)
    out, z_loss = _run(x2d, router_w1, router_b1, router_w2p, router_b2p,
                       re_w1, re_b1, re_w2, re_b2,
                       se_w1, se_b1, se_w2, se_b2, out_w, out_b,
                       interpret=interpret)
    return out.reshape(1, S, H), z_loss[0, 0]


# speedup vs baseline: 1.0818x; 1.0818x over previous
import functools

import jax
import jax.numpy as jnp
from jax import lax
from jax.experimental import pallas as pl
from jax.experimental.pallas import tpu as pltpu
from jax.experimental.pallas import tpu_sc as plsc

H = 768
D_FF = 3072
S = 2048
NR, NS = 4, 2
FBLK = 1536
NF = D_FF // FBLK
KBLK = FBLK // 2
NEG = -1e30

NWORKERS = 32
TOKW = S // NWORKERS
LANES = 16


def _gelu(x):
    return 0.5 * x * (1.0 + jax.lax.erf(x * 0.7071067811865476))


def _router_core(x_ref, w1_ref, b1_ref, w2t_ref, b2t_ref, logits_ref, z_ref):
    x = x_ref[...]
    hr = _gelu(jnp.dot(x, w1_ref[...], preferred_element_type=jnp.float32)
               + b1_ref[...])
    logits = lax.dot_general(w2t_ref[...], hr, (((1,), (1,)), ((), ())),
                             preferred_element_type=jnp.float32) + b2t_ref[...]
    row = jax.lax.broadcasted_iota(jnp.int32, logits.shape, 0)
    logits = jnp.where(row < NR, logits, NEG)
    logits_ref[...] = logits
    m1 = jnp.max(logits, axis=0)
    lse = m1 + jnp.log(jnp.sum(jnp.exp(logits - m1[None, :]), axis=0))
    z_ref[...] = jnp.mean(jnp.square(lse)).reshape(1, 1)


def _sc_topk_body(logits_hbm, wmat_hbm, lg_v, wm_v):
    wid = lax.axis_index("s") * 2 + lax.axis_index("c")
    base = wid * TOKW
    for j in range(NR):
        pltpu.sync_copy(logits_hbm.at[pl.ds(j * S + base, TOKW)], lg_v.at[j])
    for i in range(TOKW // LANES):
        lsl = pl.ds(LANES * i, LANES)
        l = [lg_v[j, lsl] for j in range(NR)]
        m1 = jnp.maximum(jnp.maximum(l[0], l[1]), jnp.maximum(l[2], l[3]))
        idx1 = jnp.where(l[0] == m1, 0,
                         jnp.where(l[1] == m1, 1,
                                   jnp.where(l[2] == m1, 2, 3)))
        l2 = [jnp.where(idx1 == j, NEG, l[j]) for j in range(NR)]
        m2 = jnp.maximum(jnp.maximum(l2[0], l2[1]),
                         jnp.maximum(l2[2], l2[3]))
        idx2 = jnp.where(l2[0] == m2, 0,
                         jnp.where(l2[1] == m2, 1,
                                   jnp.where(l2[2] == m2, 2, 3)))
        e2 = jnp.exp(m2 - m1)
        wa = 1.0 / (1.0 + e2)
        wb = e2 * wa
        sl = pl.ds(LANES * i, LANES)
        for j in range(NR):
            wm_v[j, sl] = (jnp.where(idx1 == j, wa, 0.0)
                           + jnp.where(idx2 == j, wb, 0.0))
        half = jnp.full((LANES,), 1.0 / NS, jnp.float32)
        zero = jnp.zeros((LANES,), jnp.float32)
        for j in range(NR, NR + NS):
            wm_v[j, sl] = half
        for j in range(NR + NS, 8):
            wm_v[j, sl] = zero
    for j in range(8):
        pltpu.sync_copy(wm_v.at[j], wmat_hbm.at[pl.ds(j * S + base, TOKW)])


def _ffn_chunk(x, w1_ref, b1_ref, w2_ref):
    h = _gelu(jnp.dot(x, w1_ref[0], preferred_element_type=jnp.float32)
              + b1_ref[0, 0])
    return jnp.dot(h, w2_ref[0], preferred_element_type=jnp.float32)


def _shared_body(x_ref, rw1_ref, rb1_ref, rw2t_ref, rb2t_ref,
                 w1_ref, b1_ref, w2_ref, b2_ref,
                 acc_ref, logits_ref, z_ref):
    g = pl.program_id(0)

    @pl.when(g == 0)
    def _router():
        _router_core(x_ref, rw1_ref, rb1_ref, rw2t_ref, rb2t_ref,
                     logits_ref, z_ref)

    @pl.when(g >= 1)
    def _expert():
        f = (g - 1) % NF
        delta = (1.0 / NS) * _ffn_chunk(x_ref[...], w1_ref, b1_ref, w2_ref)

        @pl.when(f == 0)
        def _bias():
            acc_ref[...] = jnp.where(g == 1, 0.0, acc_ref[...]) \
                + (1.0 / NS) * b2_ref[0, 0][None, :]

        acc_ref[...] += delta


def _routed_body(accs_ref, x_ref, wmat_ref,
                 w1_ref, b1_ref, w2_ref, b2_ref,
                 ow_ref, ob_ref, out_ref):
    g = pl.program_id(0)

    @pl.when(g == 0)
    def _init():
        out_ref[...] = accs_ref[...]

    @pl.when(g < NR * NF)
    def _expert():
        e = g // NF
        f = g % NF
        wcol = jnp.zeros((S,), jnp.float32)
        for j in range(NR):
            wcol = wcol + jnp.where(e == j, wmat_ref[j, :], 0.0)
        delta = wcol[:, None] * _ffn_chunk(x_ref[...], w1_ref, b1_ref,
                                           w2_ref)

        @pl.when(f == 0)
        def _bias():
            out_ref[...] += wcol[:, None] * b2_ref[0, 0][None, :]

        out_ref[...] += delta

    @pl.when(g == NR * NF)
    def _finish():
        o = jnp.dot(out_ref[...], ow_ref[...],
                    preferred_element_type=jnp.float32)
        o = (o + ob_ref[...]) * 0.5 + x_ref[...] * 0.5
        mean = jnp.mean(o, axis=-1, keepdims=True)
        o = o - mean
        var = jnp.mean(jnp.square(o), axis=-1, keepdims=True)
        out_ref[...] = o * jax.lax.rsqrt(var + 1e-6)


def _const_spec(shape):
    return pl.BlockSpec(shape, lambda *_: tuple(0 for _ in shape))


def _sc_topk(logits_flat):
    run = functools.partial(
        pl.kernel,
        mesh=plsc.VectorSubcoreMesh(core_axis_name="c", subcore_axis_name="s"),
        out_type=jax.ShapeDtypeStruct((8 * S,), jnp.float32),
        scratch_types=[pltpu.VMEM((NR, TOKW), jnp.float32),
                       pltpu.VMEM((8, TOKW), jnp.float32)],
    )(_sc_topk_body)
    return run(logits_flat)


@jax.jit
def _run(x2d, router_w1, router_b1, router_w2p, router_b2p,
         re_w1, re_b1, re_w2, re_b2,
         se_w1, se_b1, se_w2, se_b2, out_w, out_b):
    def sidx(g):
        gg = jnp.maximum(g - 1, 0)
        return gg // NF, gg % NF

    accs, logits, z_loss = pl.pallas_call(
        _shared_body,
        grid=(NS * NF + 1,),
        in_specs=[
            _const_spec((S, H)), _const_spec((H, H)),
            _const_spec((1, H)), _const_spec((8, H)), _const_spec((8, 1)),
            pl.BlockSpec((1, H, FBLK), lambda g: (sidx(g)[0], 0, sidx(g)[1])),
            pl.BlockSpec((1, 1, FBLK), lambda g: (sidx(g)[0], 0, sidx(g)[1])),
            pl.BlockSpec((1, FBLK, H), lambda g: (*sidx(g), 0)),
            pl.BlockSpec((1, 1, H), lambda g: (sidx(g)[0], 0, 0)),
        ],
        out_specs=[_const_spec((S, H)), _const_spec((8, S)),
                   _const_spec((1, 1))],
        out_shape=[jax.ShapeDtypeStruct((S, H), jnp.float32),
                   jax.ShapeDtypeStruct((8, S), jnp.float32),
                   jax.ShapeDtypeStruct((1, 1), jnp.float32)],
    )(x2d, router_w1, router_b1.reshape(1, H), router_w2p, router_b2p,
      se_w1, se_b1.reshape(NS, 1, D_FF), se_w2, se_b2.reshape(NS, 1, H))

    wmat = _sc_topk(logits.reshape(-1)).reshape(8, S)

    def ridx(g):
        gg = jnp.minimum(g, NR * NF - 1)
        return gg // NF, gg % NF

    out = pl.pallas_call(
        _routed_body,
        grid=(NR * NF + 1,),
        in_specs=[
            _const_spec((S, H)), _const_spec((S, H)), _const_spec((8, S)),
            pl.BlockSpec((1, H, FBLK), lambda g: (ridx(g)[0], 0, ridx(g)[1])),
            pl.BlockSpec((1, 1, FBLK), lambda g: (ridx(g)[0], 0, ridx(g)[1])),
            pl.BlockSpec((1, FBLK, H), lambda g: (*ridx(g), 0)),
            pl.BlockSpec((1, 1, H), lambda g: (ridx(g)[0], 0, 0)),
            _const_spec((H, H)), _const_spec((1, H)),
        ],
        out_specs=_const_spec((S, H)),
        out_shape=jax.ShapeDtypeStruct((S, H), jnp.float32),
    )(accs, x2d, wmat, re_w1, re_b1.reshape(NR, 1, D_FF),
      re_w2, re_b2.reshape(NR, 1, H), out_w, out_b.reshape(1, H))
    return out, z_loss


def kernel(hidden_states, router_w1, router_b1, router_w2, router_b2,
           re_w1, re_b1, re_w2, re_b2, se_w1, se_b1, se_w2, se_b2,
           out_w, out_b):
    x2d = hidden_states.reshape(S, H)
    router_w2p = jnp.pad(router_w2, ((0, 0), (0, 8 - NR))).T
    router_b2p = jnp.pad(router_b2, (0, 8 - NR)).reshape(8, 1)
    out, z_loss = _run(x2d, router_w1, router_b1, router_w2p, router_b2p,
                       re_w1, re_b1, re_w2, re_b2,
                       se_w1, se_b1, se_w2, se_b2, out_w, out_b)
    return out.reshape(1, S, H), z_loss[0, 0]
